# baseline (speedup 1.0000x reference)
import functools

import jax
import jax.numpy as jnp
from jax import lax
from jax.experimental import pallas as pl
from jax.experimental.pallas import tpu as pltpu
from jax.experimental.pallas import tpu_sc as plsc

_RADIUS = 0.1
_K = 32
_EPS = 1e-5

_TM = 256
_TN = 512
_TZ = 512
_NW = 32
_CQ = 4


def _bitmap_body(qt_ref, s_ref, bm_ref, *, tn):
    r2 = jnp.float32(_RADIUS * _RADIUS)
    qx = qt_ref[:, 0:1]
    qy = qt_ref[:, 1:2]
    qz = qt_ref[:, 2:3]
    jg = lax.broadcasted_iota(jnp.int32, (tn, tn // 16), 0)
    gg = lax.broadcasted_iota(jnp.int32, (tn, tn // 16), 1)
    pw = jnp.where((jg // 16) == gg, 1 << (jg % 16), 0).astype(jnp.float32)
    nsub = s_ref.shape[1] // tn
    bms = []
    for u in range(nsub):
        sx = s_ref[0:1, pl.ds(u * tn, tn)]
        sy = s_ref[1:2, pl.ds(u * tn, tn)]
        sz = s_ref[2:3, pl.ds(u * tn, tn)]
        d2 = (qx - sx) ** 2 + (qy - sy) ** 2 + (qz - sz) ** 2
        within = (d2 < r2).astype(jnp.float32)
        bm = jnp.dot(within, pw, preferred_element_type=jnp.float32)
        bms.append(bm.astype(jnp.int32))
    bm_ref[...] = jnp.concatenate(bms, axis=1)


def _zmat_body(a_ref, b_ref, o_ref):
    o_ref[...] = jnp.dot(a_ref[...], b_ref[...],
                         preferred_element_type=jnp.float32,
                         precision=lax.Precision.HIGHEST)


def _stats_body(h_ref, qt_ref, wx_ref, o_ref, *, n_real, tm):
    i = pl.program_id(0)

    @pl.when(i == 0)
    def _():
        o_ref[...] = jnp.zeros_like(o_ref)

    c = jnp.dot(qt_ref[...], wx_ref[...],
                preferred_element_type=jnp.float32,
                precision=lax.Precision.HIGHEST)
    h = h_ref[...] - c
    rid = i * tm + lax.broadcasted_iota(jnp.int32, (tm, 1), 0)
    m = (rid < n_real).astype(jnp.float32)
    hm = h * m
    o_ref[0:1, :] = o_ref[0:1, :] + jnp.sum(hm, axis=0, keepdims=True)
    o_ref[1:2, :] = o_ref[1:2, :] + jnp.sum(hm * h, axis=0, keepdims=True)


def _bn_body(h_ref, qt_ref, wx_ref, st_ref, g_ref, b_ref, o_ref, *, n_real):
    inv_n = jnp.float32(1.0 / n_real)
    mean = st_ref[0:1, :] * inv_n
    var = st_ref[1:2, :] * inv_n - mean * mean
    inv = lax.rsqrt(var + jnp.float32(_EPS))
    c = jnp.dot(qt_ref[...], wx_ref[...],
                preferred_element_type=jnp.float32,
                precision=lax.Precision.HIGHEST)
    h = h_ref[...] - c
    y = (h - mean) * (inv * g_ref[...]) + b_ref[...]
    o_ref[...] = jnp.maximum(y, 0.0).T


def _scgm_body(z_hbm, bm_hbm, out_hbm, words_v, wid_v, cand_v, idx_va,
               idx_vb, rows_va, rows_vb, out_v, sema, semb, *, per_w, d,
               wpq):
    cid = lax.axis_index("c")
    sid = lax.axis_index("s")
    wid = sid * 2 + cid
    ng = d // 16
    nwc = wpq // 32
    nchunks = per_w // _CQ
    iota = lax.iota(jnp.int32, 16)
    lanebit = (jnp.int32(1) << iota)
    zeros16 = jnp.zeros((16,), jnp.int32)

    def extract(ci, idx_v):
        q0 = wid * per_w + ci * _CQ
        pltpu.sync_copy(bm_hbm.at[pl.ds(q0 * wpq, _CQ * wpq)], words_v)
        for q in range(_CQ):
            def scan_c(c, tot):
                wv0 = words_v[pl.ds(q * wpq + c * 32, 16)]
                wv1 = words_v[pl.ds(q * wpq + c * 32 + 16, 16)]
                nz0 = wv0 != 0
                nz1 = wv1 != 0
                r0 = plsc.cumsum(jnp.where(nz0, 1, 0))
                r1 = plsc.cumsum(jnp.where(nz1, 1, 0))
                p0 = jnp.maximum(tot + r0 - 1, 0)
                tot0 = tot + plsc.all_reduce_population_count(nz0)
                p1 = jnp.maximum(tot0 + r1 - 1, 0)
                plsc.store_scatter(wid_v, [p0], c * 32 + iota, mask=nz0)
                plsc.store_scatter(wid_v, [p1], c * 32 + 16 + iota,
                                   mask=nz1)
                return tot0 + plsc.all_reduce_population_count(nz1)

            tot = lax.fori_loop(0, nwc, scan_c, zeros16)
            nw = jnp.max(tot)
            cand_v[pl.ds(0, 16)] = zeros16
            cand_v[pl.ds(16, 16)] = zeros16

            def consume(state):
                p, p_v, off = state
                wid_s = plsc.load_gather(wid_v, [p_v])
                w_s = plsc.load_gather(words_v, [wid_s + (q * wpq)])
                bits = (w_s & lanebit) != 0
                rank = plsc.cumsum(jnp.where(bits, 1, 0))
                pos = jnp.maximum(off + rank - 1, 0)
                sel = bits & (pos < _K)
                plsc.store_scatter(cand_v, [pos], wid_s * 16 + iota,
                                   mask=sel)
                off = off + plsc.all_reduce_population_count(bits)
                live2 = (p_v + 1) < tot
                wid_s2 = jnp.clip(plsc.load_gather(wid_v, [p_v + 1]),
                                  0, wpq - 1)
                w_s2 = plsc.load_gather(words_v, [wid_s2 + (q * wpq)])
                bits2 = (w_s2 & lanebit) != 0
                rank2 = plsc.cumsum(jnp.where(bits2, 1, 0))
                pos2 = jnp.maximum(off + rank2 - 1, 0)
                sel2 = bits2 & (pos2 < _K) & live2
                plsc.store_scatter(cand_v, [pos2], wid_s2 * 16 + iota,
                                   mask=sel2)
                pc2 = plsc.all_reduce_population_count(bits2)
                off = off + jnp.where(live2, pc2, 0)
                return p + 2, p_v + 2, off

            _, _, off = lax.while_loop(lambda s: s[0] < nw, consume,
                                       (jnp.int32(0), zeros16, zeros16))

            for h in range(_K // 16):
                slots = h * 16 + iota
                gpos = jnp.where(slots < off, slots, 0)
                idx_v[pl.ds(q * _K + h * 16, 16)] = plsc.load_gather(
                    cand_v, [gpos])

    def maxout(ci, rows_v):
        q0 = wid * per_w + ci * _CQ
        for q in range(_CQ):
            accs = [rows_v[q * _K, pl.ds(g * 16, 16)] for g in range(ng)]

            def rmax2(r, a):
                base = q * _K + 2 * r
                a2 = tuple(
                    jnp.maximum(a[g], rows_v[base + 1, pl.ds(g * 16, 16)])
                    for g in range(ng))
                return tuple(
                    jnp.maximum(a2[g], rows_v[base + 2, pl.ds(g * 16, 16)])
                    for g in range(ng))

            accs = lax.fori_loop(0, (_K - 2) // 2, rmax2, tuple(accs))
            last = q * _K + _K - 1
            for g in range(ng):
                out_v[q, pl.ds(g * 16, 16)] = jnp.maximum(
                    accs[g], rows_v[last, pl.ds(g * 16, 16)])
        pltpu.sync_copy(out_v, out_hbm.at[pl.ds(q0, _CQ)])

    extract(0, idx_va)
    pltpu.async_copy(z_hbm.at[idx_va], rows_va, sema)

    def pipe(i2, carry0):
        c0 = 2 * i2
        extract(c0 + 1, idx_vb)
        pltpu.async_copy(z_hbm.at[idx_vb], rows_vb, semb)
        pltpu.make_async_copy(z_hbm.at[pl.ds(0, _CQ * _K)], rows_va,
                              sema).wait()
        maxout(c0, rows_va)

        @pl.when(c0 + 2 < nchunks)
        def _():
            extract(c0 + 2, idx_va)
            pltpu.async_copy(z_hbm.at[idx_va], rows_va, sema)

        pltpu.make_async_copy(z_hbm.at[pl.ds(0, _CQ * _K)], rows_vb,
                              semb).wait()
        maxout(c0 + 1, rows_vb)
        return carry0

    lax.fori_loop(0, nchunks // 2, pipe, 0)


def kernel(x, xyz, W, gamma, beta):
    b, d_in, n = x.shape
    d_out = W.shape[0]
    m_pad = 10240 if n <= 10240 else ((n + 1023) // 1024) * 1024
    n_pad = m_pad
    wpq = n_pad // 16

    xyz2 = xyz[0]
    qt = jnp.full((m_pad, 4), 1e6, jnp.float32)
    qt = qt.at[:n, 0:3].set(xyz2.T)
    src = jnp.full((4, n_pad), -1e6, jnp.float32)
    src = src.at[0:3, :n].set(xyz2)

    tb = _TN * 4
    mh0 = 2048
    halves = [(0, mh0), (mh0, m_pad - mh0)]

    def bitmap_half(q_lo, mh):
        return pl.pallas_call(
            functools.partial(_bitmap_body, tn=_TN),
            grid=(mh // _TM, n_pad // tb),
            in_specs=[
                pl.BlockSpec((_TM, 4), lambda i, t: (i, 0)),
                pl.BlockSpec((4, tb), lambda i, t: (0, t)),
            ],
            out_specs=pl.BlockSpec((_TM, tb // 16), lambda i, t: (i, t)),
            out_shape=jax.ShapeDtypeStruct((mh, wpq), jnp.int32),
        )(qt[q_lo:q_lo + mh], src)

    bms = [bitmap_half(q_lo, mh) for q_lo, mh in halves]

    kdim = d_in + 8
    xa = jnp.zeros((n_pad, kdim), jnp.float32)
    xa = xa.at[:n, :d_in].set(x[0].T)
    xa = xa.at[:n, d_in:d_in + 3].set(xyz2.T / _RADIUS)
    wt = jnp.zeros((kdim, d_out), jnp.float32)
    wt = wt.at[:d_in + 3, :].set(W.T)
    zT = pl.pallas_call(
        _zmat_body,
        grid=(n_pad // _TZ,),
        in_specs=[
            pl.BlockSpec((_TZ, kdim), lambda i: (i, 0)),
            pl.BlockSpec((kdim, d_out), lambda i: (0, 0)),
        ],
        out_specs=pl.BlockSpec((_TZ, d_out), lambda i: (i, 0)),
        out_shape=jax.ShapeDtypeStruct((n_pad, d_out), jnp.float32),
    )(xa, wt)

    def sc_half(bm_h, mh):
        per_w = mh // _NW
        gm = functools.partial(
            pl.kernel,
            mesh=plsc.VectorSubcoreMesh(core_axis_name="c",
                                        subcore_axis_name="s"),
            compiler_params=pltpu.CompilerParams(needs_layout_passes=False),
            out_type=jax.ShapeDtypeStruct((mh, d_out), jnp.float32),
            scratch_types=[
                pltpu.VMEM((_CQ * wpq,), jnp.int32),
                pltpu.VMEM((wpq,), jnp.int32),
                pltpu.VMEM((_K,), jnp.int32),
                pltpu.VMEM((_CQ * _K,), jnp.int32),
                pltpu.VMEM((_CQ * _K,), jnp.int32),
                pltpu.VMEM((_CQ * _K, d_out), jnp.float32),
                pltpu.VMEM((_CQ * _K, d_out), jnp.float32),
                pltpu.VMEM((_CQ, d_out), jnp.float32),
                pltpu.SemaphoreType.DMA,
                pltpu.SemaphoreType.DMA,
            ],
        )(functools.partial(_scgm_body, per_w=per_w, d=d_out, wpq=wpq))
        return gm(zT, bm_h.reshape(mh * wpq))

    hmaxs = [sc_half(bm_h, mh) for bm_h, (_, mh) in zip(bms, halves)]

    wx = jnp.zeros((4, d_out), jnp.float32)
    wx = wx.at[0:3, :].set(W[:, d_in:d_in + 3].T / _RADIUS)

    def stats_half(hmax_h, q_lo, mh):
        return pl.pallas_call(
            functools.partial(_stats_body, n_real=n - q_lo, tm=_TM),
            grid=(mh // _TM,),
            in_specs=[
                pl.BlockSpec((_TM, d_out), lambda i: (i, 0)),
                pl.BlockSpec((_TM, 4), lambda i: (i, 0)),
                pl.BlockSpec((4, d_out), lambda i: (0, 0)),
            ],
            out_specs=pl.BlockSpec((8, d_out), lambda i: (0, 0)),
            out_shape=jax.ShapeDtypeStruct((8, d_out), jnp.float32),
        )(hmax_h, qt[q_lo:q_lo + mh], wx)

    stats = sum(stats_half(h, q_lo, mh)
                for h, (q_lo, mh) in zip(hmaxs, halves))

    def bn_half(hmax_h, q_lo, mh):
        return pl.pallas_call(
            functools.partial(_bn_body, n_real=n),
            grid=(mh // _TM,),
            in_specs=[
                pl.BlockSpec((_TM, d_out), lambda i: (i, 0)),
                pl.BlockSpec((_TM, 4), lambda i: (i, 0)),
                pl.BlockSpec((4, d_out), lambda i: (0, 0)),
                pl.BlockSpec((8, d_out), lambda i: (0, 0)),
                pl.BlockSpec((1, d_out), lambda i: (0, 0)),
                pl.BlockSpec((1, d_out), lambda i: (0, 0)),
            ],
            out_specs=pl.BlockSpec((d_out, _TM), lambda i: (0, i)),
            out_shape=jax.ShapeDtypeStruct((d_out, mh), jnp.float32),
        )(hmax_h, qt[q_lo:q_lo + mh], wx, stats, gamma[None, :],
          beta[None, :])

    outT = jnp.concatenate(
        [bn_half(h, q_lo, mh) for h, (q_lo, mh) in zip(hmaxs, halves)],
        axis=1)
    return outT[:, :n][None]


# speedup vs baseline: 1.0145x; 1.0145x over previous
import functools

import jax
import jax.numpy as jnp
from jax import lax
from jax.experimental import pallas as pl
from jax.experimental.pallas import tpu as pltpu
from jax.experimental.pallas import tpu_sc as plsc

_RADIUS = 0.1
_K = 32
_EPS = 1e-5

_TM = 256
_TN = 512
_TZ = 512
_NW = 32
_CQ = 4


def _bitmap_body(qt_ref, s_ref, bm_ref, *, tn):
    r2 = jnp.float32(_RADIUS * _RADIUS)
    qx = qt_ref[:, 0:1]
    qy = qt_ref[:, 1:2]
    qz = qt_ref[:, 2:3]
    jg = lax.broadcasted_iota(jnp.int32, (tn, tn // 16), 0)
    gg = lax.broadcasted_iota(jnp.int32, (tn, tn // 16), 1)
    pw = jnp.where((jg // 16) == gg, 1 << (jg % 16), 0).astype(jnp.float32)
    nsub = s_ref.shape[1] // tn
    bms = []
    for u in range(nsub):
        sx = s_ref[0:1, pl.ds(u * tn, tn)]
        sy = s_ref[1:2, pl.ds(u * tn, tn)]
        sz = s_ref[2:3, pl.ds(u * tn, tn)]
        d2 = (qx - sx) ** 2 + (qy - sy) ** 2 + (qz - sz) ** 2
        within = (d2 < r2).astype(jnp.float32)
        bm = jnp.dot(within, pw, preferred_element_type=jnp.float32)
        bms.append(bm.astype(jnp.int32))
    bm_ref[...] = jnp.concatenate(bms, axis=1)


def _zmat_body(a_ref, b_ref, o_ref):
    o_ref[...] = jnp.dot(a_ref[...], b_ref[...],
                         preferred_element_type=jnp.float32,
                         precision=lax.Precision.HIGHEST)


def _stats_body(h_ref, qt_ref, wx_ref, o_ref, *, n_real, tm):
    i = pl.program_id(0)

    @pl.when(i == 0)
    def _():
        o_ref[...] = jnp.zeros_like(o_ref)

    c = jnp.dot(qt_ref[...], wx_ref[...],
                preferred_element_type=jnp.float32,
                precision=lax.Precision.HIGHEST)
    h = h_ref[...] - c
    rid = i * tm + lax.broadcasted_iota(jnp.int32, (tm, 1), 0)
    m = (rid < n_real).astype(jnp.float32)
    hm = h * m
    o_ref[0:1, :] = o_ref[0:1, :] + jnp.sum(hm, axis=0, keepdims=True)
    o_ref[1:2, :] = o_ref[1:2, :] + jnp.sum(hm * h, axis=0, keepdims=True)


def _bn_body(h_ref, qt_ref, wx_ref, st_ref, g_ref, b_ref, o_ref, *, n_real):
    inv_n = jnp.float32(1.0 / n_real)
    mean = st_ref[0:1, :] * inv_n
    var = st_ref[1:2, :] * inv_n - mean * mean
    inv = lax.rsqrt(var + jnp.float32(_EPS))
    c = jnp.dot(qt_ref[...], wx_ref[...],
                preferred_element_type=jnp.float32,
                precision=lax.Precision.HIGHEST)
    h = h_ref[...] - c
    y = (h - mean) * (inv * g_ref[...]) + b_ref[...]
    o_ref[...] = jnp.maximum(y, 0.0).T


def _scgm_body(z_hbm, bm_hbm, out_hbm, words_v, wid_v, cand_v, idx_va,
               idx_vb, rows_va, rows_vb, out_v, sema, semb, *, per_w, d,
               wpq):
    cid = lax.axis_index("c")
    sid = lax.axis_index("s")
    wid = sid * 2 + cid
    ng = d // 16
    nwc = wpq // 32
    nchunks = per_w // _CQ
    iota = lax.iota(jnp.int32, 16)
    lanebit = (jnp.int32(1) << iota)
    zeros16 = jnp.zeros((16,), jnp.int32)

    def extract(ci, idx_v):
        q0 = wid * per_w + ci * _CQ
        pltpu.sync_copy(bm_hbm.at[pl.ds(q0 * wpq, _CQ * wpq)], words_v)
        for q in range(_CQ):
            def scan_c(c, tot):
                wv0 = words_v[pl.ds(q * wpq + c * 32, 16)]
                wv1 = words_v[pl.ds(q * wpq + c * 32 + 16, 16)]
                nz0 = wv0 != 0
                nz1 = wv1 != 0
                r0 = plsc.cumsum(jnp.where(nz0, 1, 0))
                r1 = plsc.cumsum(jnp.where(nz1, 1, 0))
                p0 = jnp.maximum(tot + r0 - 1, 0)
                tot0 = tot + plsc.all_reduce_population_count(nz0)
                p1 = jnp.maximum(tot0 + r1 - 1, 0)
                plsc.store_scatter(wid_v, [p0], c * 32 + iota, mask=nz0)
                plsc.store_scatter(wid_v, [p1], c * 32 + 16 + iota,
                                   mask=nz1)
                return tot0 + plsc.all_reduce_population_count(nz1)

            tot = lax.fori_loop(0, nwc, scan_c, zeros16)
            nw = jnp.max(tot)
            cand_v[pl.ds(0, 16)] = zeros16
            cand_v[pl.ds(16, 16)] = zeros16

            def consume(state):
                p, p_v, off = state
                wid_s = plsc.load_gather(wid_v, [p_v])
                w_s = plsc.load_gather(words_v, [wid_s + (q * wpq)])
                bits = (w_s & lanebit) != 0
                rank = plsc.cumsum(jnp.where(bits, 1, 0))
                pos = jnp.maximum(off + rank - 1, 0)
                sel = bits & (pos < _K)
                plsc.store_scatter(cand_v, [pos], wid_s * 16 + iota,
                                   mask=sel)
                return (p + 1, p_v + 1,
                        off + plsc.all_reduce_population_count(bits))

            _, _, off = lax.while_loop(lambda s: s[0] < nw, consume,
                                       (jnp.int32(0), zeros16, zeros16))

            for h in range(_K // 16):
                slots = h * 16 + iota
                gpos = jnp.where(slots < off, slots, 0)
                idx_v[pl.ds(q * _K + h * 16, 16)] = plsc.load_gather(
                    cand_v, [gpos])

    def maxout(ci, rows_v):
        q0 = wid * per_w + ci * _CQ
        for q in range(_CQ):
            accs = [rows_v[q * _K, pl.ds(g * 16, 16)] for g in range(ng)]

            def rmax2(r, a):
                base = q * _K + 2 * r
                a2 = tuple(
                    jnp.maximum(a[g], rows_v[base + 1, pl.ds(g * 16, 16)])
                    for g in range(ng))
                return tuple(
                    jnp.maximum(a2[g], rows_v[base + 2, pl.ds(g * 16, 16)])
                    for g in range(ng))

            accs = lax.fori_loop(0, (_K - 2) // 2, rmax2, tuple(accs))
            last = q * _K + _K - 1
            for g in range(ng):
                out_v[q, pl.ds(g * 16, 16)] = jnp.maximum(
                    accs[g], rows_v[last, pl.ds(g * 16, 16)])
        pltpu.sync_copy(out_v, out_hbm.at[pl.ds(q0, _CQ)])

    extract(0, idx_va)
    pltpu.async_copy(z_hbm.at[idx_va], rows_va, sema)

    def pipe(i2, carry0):
        c0 = 2 * i2
        extract(c0 + 1, idx_vb)
        pltpu.async_copy(z_hbm.at[idx_vb], rows_vb, semb)
        pltpu.make_async_copy(z_hbm.at[pl.ds(0, _CQ * _K)], rows_va,
                              sema).wait()
        maxout(c0, rows_va)

        @pl.when(c0 + 2 < nchunks)
        def _():
            extract(c0 + 2, idx_va)
            pltpu.async_copy(z_hbm.at[idx_va], rows_va, sema)

        pltpu.make_async_copy(z_hbm.at[pl.ds(0, _CQ * _K)], rows_vb,
                              semb).wait()
        maxout(c0 + 1, rows_vb)
        return carry0

    lax.fori_loop(0, nchunks // 2, pipe, 0)


def kernel(x, xyz, W, gamma, beta):
    b, d_in, n = x.shape
    d_out = W.shape[0]
    m_pad = 10240 if n <= 10240 else ((n + 1023) // 1024) * 1024
    n_pad = m_pad
    wpq = n_pad // 16

    xyz2 = xyz[0]
    qt = jnp.full((m_pad, 4), 1e6, jnp.float32)
    qt = qt.at[:n, 0:3].set(xyz2.T)
    src = jnp.full((4, n_pad), -1e6, jnp.float32)
    src = src.at[0:3, :n].set(xyz2)

    tb = _TN * 4
    mh0 = 2048
    halves = [(0, mh0), (mh0, m_pad - mh0)]

    def bitmap_half(q_lo, mh):
        return pl.pallas_call(
            functools.partial(_bitmap_body, tn=_TN),
            grid=(mh // _TM, n_pad // tb),
            in_specs=[
                pl.BlockSpec((_TM, 4), lambda i, t: (i, 0)),
                pl.BlockSpec((4, tb), lambda i, t: (0, t)),
            ],
            out_specs=pl.BlockSpec((_TM, tb // 16), lambda i, t: (i, t)),
            out_shape=jax.ShapeDtypeStruct((mh, wpq), jnp.int32),
        )(qt[q_lo:q_lo + mh], src)

    bms = [bitmap_half(q_lo, mh) for q_lo, mh in halves]

    kdim = d_in + 8
    xa = jnp.zeros((n_pad, kdim), jnp.float32)
    xa = xa.at[:n, :d_in].set(x[0].T)
    xa = xa.at[:n, d_in:d_in + 3].set(xyz2.T / _RADIUS)
    wt = jnp.zeros((kdim, d_out), jnp.float32)
    wt = wt.at[:d_in + 3, :].set(W.T)
    zT = pl.pallas_call(
        _zmat_body,
        grid=(n_pad // _TZ,),
        in_specs=[
            pl.BlockSpec((_TZ, kdim), lambda i: (i, 0)),
            pl.BlockSpec((kdim, d_out), lambda i: (0, 0)),
        ],
        out_specs=pl.BlockSpec((_TZ, d_out), lambda i: (i, 0)),
        out_shape=jax.ShapeDtypeStruct((n_pad, d_out), jnp.float32),
    )(xa, wt)

    def sc_half(bm_h, mh):
        per_w = mh // _NW
        gm = functools.partial(
            pl.kernel,
            mesh=plsc.VectorSubcoreMesh(core_axis_name="c",
                                        subcore_axis_name="s"),
            compiler_params=pltpu.CompilerParams(needs_layout_passes=False),
            out_type=jax.ShapeDtypeStruct((mh, d_out), jnp.float32),
            scratch_types=[
                pltpu.VMEM((_CQ * wpq,), jnp.int32),
                pltpu.VMEM((wpq,), jnp.int32),
                pltpu.VMEM((_K,), jnp.int32),
                pltpu.VMEM((_CQ * _K,), jnp.int32),
                pltpu.VMEM((_CQ * _K,), jnp.int32),
                pltpu.VMEM((_CQ * _K, d_out), jnp.float32),
                pltpu.VMEM((_CQ * _K, d_out), jnp.float32),
                pltpu.VMEM((_CQ, d_out), jnp.float32),
                pltpu.SemaphoreType.DMA,
                pltpu.SemaphoreType.DMA,
            ],
        )(functools.partial(_scgm_body, per_w=per_w, d=d_out, wpq=wpq))
        return gm(zT, bm_h.reshape(mh * wpq))

    hmaxs = [sc_half(bm_h, mh) for bm_h, (_, mh) in zip(bms, halves)]

    wx = jnp.zeros((4, d_out), jnp.float32)
    wx = wx.at[0:3, :].set(W[:, d_in:d_in + 3].T / _RADIUS)

    def stats_half(hmax_h, q_lo, mh):
        return pl.pallas_call(
            functools.partial(_stats_body, n_real=n - q_lo, tm=_TM),
            grid=(mh // _TM,),
            in_specs=[
                pl.BlockSpec((_TM, d_out), lambda i: (i, 0)),
                pl.BlockSpec((_TM, 4), lambda i: (i, 0)),
                pl.BlockSpec((4, d_out), lambda i: (0, 0)),
            ],
            out_specs=pl.BlockSpec((8, d_out), lambda i: (0, 0)),
            out_shape=jax.ShapeDtypeStruct((8, d_out), jnp.float32),
        )(hmax_h, qt[q_lo:q_lo + mh], wx)

    stats = sum(stats_half(h, q_lo, mh)
                for h, (q_lo, mh) in zip(hmaxs, halves))

    def bn_half(hmax_h, q_lo, mh):
        return pl.pallas_call(
            functools.partial(_bn_body, n_real=n),
            grid=(mh // _TM,),
            in_specs=[
                pl.BlockSpec((_TM, d_out), lambda i: (i, 0)),
                pl.BlockSpec((_TM, 4), lambda i: (i, 0)),
                pl.BlockSpec((4, d_out), lambda i: (0, 0)),
                pl.BlockSpec((8, d_out), lambda i: (0, 0)),
                pl.BlockSpec((1, d_out), lambda i: (0, 0)),
                pl.BlockSpec((1, d_out), lambda i: (0, 0)),
            ],
            out_specs=pl.BlockSpec((d_out, _TM), lambda i: (0, i)),
            out_shape=jax.ShapeDtypeStruct((d_out, mh), jnp.float32),
        )(hmax_h, qt[q_lo:q_lo + mh], wx, stats, gamma[None, :],
          beta[None, :])

    outT = jnp.concatenate(
        [bn_half(h, q_lo, mh) for h, (q_lo, mh) in zip(hmaxs, halves)],
        axis=1)
    return outT[:, :n][None]
